# baseline (speedup 1.0000x reference)
import jax
import jax.numpy as jnp
from jax.experimental import pallas as pl
from jax.experimental.pallas import tpu as pltpu

_S = 32
_E = 96
_L = 128
_H = 512
_B = 2048
_M = 64
_C = 8

_BM_LIFT = 4096
_R = 512


def _encode(x, w1, b1, w2, b2, w3, b3, wo, bo):
    h = jnp.maximum(jnp.dot(x, w1, preferred_element_type=jnp.float32) + b1, 0.0)
    h = jnp.maximum(jnp.dot(h, w2, preferred_element_type=jnp.float32) + b2, 0.0)
    h = jnp.maximum(jnp.dot(h, w3, preferred_element_type=jnp.float32) + b3, 0.0)
    return jnp.dot(h, wo, preferred_element_type=jnp.float32) + bo


def _lift_body(x_ref, w1, b1, w2, b2, w3, b3, wo, bo, out_ref):
    x = x_ref[...]
    e = _encode(x, w1[...], b1[...], w2[...], b2[...], w3[...], b3[...],
                wo[...], bo[...])
    out_ref[...] = jnp.concatenate([x, e], axis=-1)


_DN_C0 = (((0,), (0,)), ((), ()))


def _koop_body(x_ref, ut_ref, eye_ref, w1, b1, w2, b2, w3, b3, wo, bo, a_ref,
               bm_ref, out_ref, outx_ref):
    x = x_ref[...]
    e = _encode(x, w1[...], b1[...], w2[...], b2[...], w3[...], b3[...],
                wo[...], bo[...])
    z = jnp.concatenate([x, e], axis=-1)
    a = a_ref[...]
    bm = bm_ref[...]
    eye = eye_ref[...]
    for t in range(_M):
        ut_t = ut_ref[t * _C:(t + 1) * _C, :]
        bu = jax.lax.dot_general(ut_t, bm, _DN_C0,
                                 preferred_element_type=jnp.float32)
        z = jnp.dot(z, a, preferred_element_type=jnp.float32) + bu
        out_ref[:, t * _L:(t + 1) * _L] = z
        outx_ref[t * _S:(t + 1) * _S, :] = jax.lax.dot_general(
            z[:, :_S], eye, _DN_C0, preferred_element_type=jnp.float32,
            precision=jax.lax.Precision.HIGHEST)


def kernel(x_k, u_seq, x_next_seq, W1, b1, W2, b2, W3, b3, Wo, bo, A, Bmat):
    f32 = jnp.float32
    b1r, b2r, b3r, bor = (b.reshape(1, -1) for b in (b1, b2, b3, bo))
    wspecs = [
        pl.BlockSpec((_S, _H), lambda *i: (0, 0)),
        pl.BlockSpec((1, _H), lambda *i: (0, 0)),
        pl.BlockSpec((_H, _H), lambda *i: (0, 0)),
        pl.BlockSpec((1, _H), lambda *i: (0, 0)),
        pl.BlockSpec((_H, _H), lambda *i: (0, 0)),
        pl.BlockSpec((1, _H), lambda *i: (0, 0)),
        pl.BlockSpec((_H, _E), lambda *i: (0, 0)),
        pl.BlockSpec((1, _E), lambda *i: (0, 0)),
    ]
    weights = (W1, b1r, W2, b2r, W3, b3r, Wo, bor)

    nrows = _B * _M
    nblk = nrows // _BM_LIFT
    x_flat = x_next_seq.reshape(nrows, _S)
    z_target_flat = pl.pallas_call(
        _lift_body,
        grid=(nblk,),
        in_specs=[pl.BlockSpec((_BM_LIFT, _S), lambda i: (i, 0))] + wspecs,
        out_specs=pl.BlockSpec((_BM_LIFT, _L), lambda i: (i, 0)),
        out_shape=jax.ShapeDtypeStruct((nrows, _L), f32),
        compiler_params=pltpu.CompilerParams(
            dimension_semantics=("arbitrary",),
            vmem_limit_bytes=56 * 1024 * 1024,
        ),
        name="mlp_lift",
    )(x_flat, *weights)
    z_target_seq = z_target_flat.reshape(_B, _M, _L)

    u_t_flat = jnp.transpose(u_seq, (1, 2, 0)).reshape(_M * _C, _B)
    eye = jnp.eye(_R, dtype=f32)
    z_pred_flat, x_pred_t = pl.pallas_call(
        _koop_body,
        grid=(_B // _R,),
        in_specs=[
            pl.BlockSpec((_R, _S), lambda i: (i, 0)),
            pl.BlockSpec((_M * _C, _R), lambda i: (0, i)),
            pl.BlockSpec((_R, _R), lambda i: (0, 0)),
        ] + wspecs + [
            pl.BlockSpec((_L, _L), lambda i: (0, 0)),
            pl.BlockSpec((_C, _L), lambda i: (0, 0)),
        ],
        out_specs=[
            pl.BlockSpec((_R, _M * _L), lambda i: (i, 0)),
            pl.BlockSpec((_M * _S, _R), lambda i: (0, i)),
        ],
        out_shape=[
            jax.ShapeDtypeStruct((_B, _M * _L), f32),
            jax.ShapeDtypeStruct((_M * _S, _B), f32),
        ],
        compiler_params=pltpu.CompilerParams(
            dimension_semantics=("arbitrary",),
            vmem_limit_bytes=56 * 1024 * 1024,
        ),
        name="koopman_recurrence",
    )(x_k, u_t_flat, eye, *weights, A, Bmat)
    z_pred_seq = z_pred_flat.reshape(_B, _M, _L)
    x_pred_seq = jnp.transpose(x_pred_t.reshape(_M, _S, _B), (2, 0, 1))
    return (z_pred_seq, x_pred_seq, z_target_seq)


# speedup vs baseline: 1.3087x; 1.3087x over previous
import jax
import jax.numpy as jnp
from jax.experimental import pallas as pl
from jax.experimental.pallas import tpu as pltpu

_S = 32
_E = 96
_L = 128
_H = 512
_B = 2048
_M = 64
_C = 8

_BM_LIFT = 4096
_R = 512


def _encode(x, w1, b1, w2, b2, w3, b3, wo, bo):
    h = jnp.maximum(jnp.dot(x, w1, preferred_element_type=jnp.float32) + b1, 0.0)
    h = jnp.maximum(jnp.dot(h, w2, preferred_element_type=jnp.float32) + b2, 0.0)
    h = jnp.maximum(jnp.dot(h, w3, preferred_element_type=jnp.float32) + b3, 0.0)
    return jnp.dot(h, wo, preferred_element_type=jnp.float32) + bo


def _lift_body(x_ref, w1, b1, w2, b2, w3, b3, wo, bo, out_ref):
    x = x_ref[...]
    e = _encode(x, w1[...], b1[...], w2[...], b2[...], w3[...], b3[...],
                wo[...], bo[...])
    out_ref[...] = jnp.concatenate([x, e], axis=-1)


_DN_C0 = (((0,), (0,)), ((), ()))


def _koop_body(x_ref, ut_ref, w1, b1, w2, b2, w3, b3, wo, bo, a_ref,
               bm_ref, out_ref):
    x = x_ref[...]
    e = _encode(x, w1[...], b1[...], w2[...], b2[...], w3[...], b3[...],
                wo[...], bo[...])
    z = jnp.concatenate([x, e], axis=-1)
    a = a_ref[...]
    bm = bm_ref[...]
    for t in range(_M):
        ut_t = ut_ref[t * _C:(t + 1) * _C, :]
        bu = jax.lax.dot_general(ut_t, bm, _DN_C0,
                                 preferred_element_type=jnp.float32)
        z = jnp.dot(z, a, preferred_element_type=jnp.float32) + bu
        out_ref[:, t * _L:(t + 1) * _L] = z


def kernel(x_k, u_seq, x_next_seq, W1, b1, W2, b2, W3, b3, Wo, bo, A, Bmat):
    f32 = jnp.float32
    b1r, b2r, b3r, bor = (b.reshape(1, -1) for b in (b1, b2, b3, bo))
    wspecs = [
        pl.BlockSpec((_S, _H), lambda *i: (0, 0)),
        pl.BlockSpec((1, _H), lambda *i: (0, 0)),
        pl.BlockSpec((_H, _H), lambda *i: (0, 0)),
        pl.BlockSpec((1, _H), lambda *i: (0, 0)),
        pl.BlockSpec((_H, _H), lambda *i: (0, 0)),
        pl.BlockSpec((1, _H), lambda *i: (0, 0)),
        pl.BlockSpec((_H, _E), lambda *i: (0, 0)),
        pl.BlockSpec((1, _E), lambda *i: (0, 0)),
    ]
    weights = (W1, b1r, W2, b2r, W3, b3r, Wo, bor)

    nrows = _B * _M
    nblk = nrows // _BM_LIFT
    x_flat = x_next_seq.reshape(nrows, _S)
    z_target_flat = pl.pallas_call(
        _lift_body,
        grid=(nblk,),
        in_specs=[pl.BlockSpec((_BM_LIFT, _S), lambda i: (i, 0))] + wspecs,
        out_specs=pl.BlockSpec((_BM_LIFT, _L), lambda i: (i, 0)),
        out_shape=jax.ShapeDtypeStruct((nrows, _L), f32),
        compiler_params=pltpu.CompilerParams(
            dimension_semantics=("arbitrary",),
            vmem_limit_bytes=56 * 1024 * 1024,
        ),
        name="mlp_lift",
    )(x_flat, *weights)
    z_target_seq = z_target_flat.reshape(_B, _M, _L)

    u_t_flat = jnp.transpose(u_seq, (1, 2, 0)).reshape(_M * _C, _B)
    z_pred_flat = pl.pallas_call(
        _koop_body,
        grid=(_B // _R,),
        in_specs=[
            pl.BlockSpec((_R, _S), lambda i: (i, 0)),
            pl.BlockSpec((_M * _C, _R), lambda i: (0, i)),
        ] + wspecs + [
            pl.BlockSpec((_L, _L), lambda i: (0, 0)),
            pl.BlockSpec((_C, _L), lambda i: (0, 0)),
        ],
        out_specs=pl.BlockSpec((_R, _M * _L), lambda i: (i, 0)),
        out_shape=jax.ShapeDtypeStruct((_B, _M * _L), f32),
        compiler_params=pltpu.CompilerParams(
            dimension_semantics=("arbitrary",),
            vmem_limit_bytes=56 * 1024 * 1024,
        ),
        name="koopman_recurrence",
    )(x_k, u_t_flat, *weights, A, Bmat)
    z_pred_seq = z_pred_flat.reshape(_B, _M, _L)
    x_pred_seq = z_pred_seq[..., :_S]
    return (z_pred_seq, x_pred_seq, z_target_seq)


# speedup vs baseline: 1.4542x; 1.1111x over previous
import jax
import jax.numpy as jnp
from jax.experimental import pallas as pl
from jax.experimental.pallas import tpu as pltpu

_S = 32
_E = 96
_L = 128
_H = 512
_B = 2048
_M = 64
_C = 8

_BM_LIFT = 4096
_R = 1024


def _encode(x, w1, b1, w2, b2, w3, b3, wo, bo):
    h = jnp.maximum(jnp.dot(x, w1, preferred_element_type=jnp.float32) + b1, 0.0)
    h = jnp.maximum(jnp.dot(h, w2, preferred_element_type=jnp.float32) + b2, 0.0)
    h = jnp.maximum(jnp.dot(h, w3, preferred_element_type=jnp.float32) + b3, 0.0)
    return jnp.dot(h, wo, preferred_element_type=jnp.float32) + bo


def _lift_body(x_ref, w1, b1, w2, b2, w3, b3, wo, bo, out_ref):
    x = x_ref[...]
    e = _encode(x, w1[...], b1[...], w2[...], b2[...], w3[...], b3[...],
                wo[...], bo[...])
    out_ref[...] = jnp.concatenate([x, e], axis=-1)


_DN_C0 = (((0,), (0,)), ((), ()))


def _koop_body(x_ref, ut_ref, w1, b1, w2, b2, w3, b3, wo, bo, a_ref,
               bm_ref, out_ref, stage_ref, sems):
    i = pl.program_id(0)
    x = x_ref[...]
    e = _encode(x, w1[...], b1[...], w2[...], b2[...], w3[...], b3[...],
                wo[...], bo[...])
    z = jnp.concatenate([x, e], axis=-1)
    a = a_ref[...]
    bm = bm_ref[...]
    rows = pl.ds(i * _R, _R)
    for t in range(_M):
        ut_t = ut_ref[t * _C:(t + 1) * _C, :]
        bu = jax.lax.dot_general(ut_t, bm, _DN_C0,
                                 preferred_element_type=jnp.float32)
        z = jnp.dot(z, a, preferred_element_type=jnp.float32) + bu
        slot = t % 2
        cp_prev = pltpu.make_async_copy(stage_ref.at[slot], out_ref.at[rows, t],
                                        sems.at[slot])
        if t >= 2:
            cp_prev.wait()
        else:
            @pl.when(i > 0)
            def _wait():
                cp_prev.wait()
        stage_ref[slot] = z
        pltpu.make_async_copy(stage_ref.at[slot], out_ref.at[rows, t],
                              sems.at[slot]).start()
    @pl.when(i == pl.num_programs(0) - 1)
    def _drain():
        for slot in range(2):
            pltpu.make_async_copy(stage_ref.at[slot], out_ref.at[rows, _M - 2 + slot],
                                  sems.at[slot]).wait()


def kernel(x_k, u_seq, x_next_seq, W1, b1, W2, b2, W3, b3, Wo, bo, A, Bmat):
    f32 = jnp.float32
    b1r, b2r, b3r, bor = (b.reshape(1, -1) for b in (b1, b2, b3, bo))
    wspecs = [
        pl.BlockSpec((_S, _H), lambda *i: (0, 0)),
        pl.BlockSpec((1, _H), lambda *i: (0, 0)),
        pl.BlockSpec((_H, _H), lambda *i: (0, 0)),
        pl.BlockSpec((1, _H), lambda *i: (0, 0)),
        pl.BlockSpec((_H, _H), lambda *i: (0, 0)),
        pl.BlockSpec((1, _H), lambda *i: (0, 0)),
        pl.BlockSpec((_H, _E), lambda *i: (0, 0)),
        pl.BlockSpec((1, _E), lambda *i: (0, 0)),
    ]
    weights = (W1, b1r, W2, b2r, W3, b3r, Wo, bor)

    nrows = _B * _M
    nblk = nrows // _BM_LIFT
    x_flat = x_next_seq.reshape(nrows, _S)
    z_target_flat = pl.pallas_call(
        _lift_body,
        grid=(nblk,),
        in_specs=[pl.BlockSpec((_BM_LIFT, _S), lambda i: (i, 0))] + wspecs,
        out_specs=pl.BlockSpec((_BM_LIFT, _L), lambda i: (i, 0)),
        out_shape=jax.ShapeDtypeStruct((nrows, _L), f32),
        compiler_params=pltpu.CompilerParams(
            dimension_semantics=("arbitrary",),
            vmem_limit_bytes=56 * 1024 * 1024,
        ),
        name="mlp_lift",
    )(x_flat, *weights)
    z_target_seq = z_target_flat.reshape(_B, _M, _L)

    u_t_flat = jnp.transpose(u_seq, (1, 2, 0)).reshape(_M * _C, _B)
    z_pred_flat = pl.pallas_call(
        _koop_body,
        grid=(_B // _R,),
        in_specs=[
            pl.BlockSpec((_R, _S), lambda i: (i, 0)),
            pl.BlockSpec((_M * _C, _R), lambda i: (0, i)),
        ] + wspecs + [
            pl.BlockSpec((_L, _L), lambda i: (0, 0)),
            pl.BlockSpec((_C, _L), lambda i: (0, 0)),
        ],
        out_specs=pl.BlockSpec(memory_space=pl.ANY),
        out_shape=jax.ShapeDtypeStruct((_B, _M, _L), f32),
        scratch_shapes=[pltpu.VMEM((2, _R, _L), f32),
                        pltpu.SemaphoreType.DMA((2,))],
        compiler_params=pltpu.CompilerParams(
            dimension_semantics=("arbitrary",),
            vmem_limit_bytes=56 * 1024 * 1024,
        ),
        name="koopman_recurrence",
    )(x_k, u_t_flat, *weights, A, Bmat)
    z_pred_seq = z_pred_flat
    x_pred_seq = z_pred_seq[..., :_S]
    return (z_pred_seq, x_pred_seq, z_target_seq)
